# Initial kernel scaffold; baseline (speedup 1.0000x reference)
import jax
import jax.numpy as jnp
from jax.experimental import pallas as pl


def kernel(input, adj, W_mean, W1, W2):
    raise NotImplementedError("write your pallas kernel here")


# speedup vs baseline: 6.5175x; 6.5175x over previous
import numpy as np
import jax
import jax.numpy as jnp
from jax.experimental import pallas as pl

OUT_F = 64
HID = 16
TRI = OUT_F * (OUT_F + 1) // 2
TRI_PAD = ((TRI + 127) // 128) * 128
THRESH = 0.005

_tri_rows, _tri_cols = np.tril_indices(OUT_F)
_t = np.arange(TRI)

_COL_OH = np.zeros((OUT_F, TRI_PAD), np.float32)
_COL_OH[_tri_cols, _t] = 1.0
_ROW_OH = np.zeros((TRI_PAD, OUT_F), np.float32)
_ROW_OH[_t, _tri_rows] = 1.0
_DIAG = np.zeros((1, TRI_PAD), np.float32)
_DIAG[0, _t[_tri_rows == _tri_cols]] = 1.0


def _feat_kernel(x_ref, w_ref, o_ref):
    o_ref[...] = jnp.dot(x_ref[...], w_ref[...],
                         preferred_element_type=jnp.float32)


def _pass1_kernel(adj_ref, ab_ref, o_ref):
    acc = jnp.dot(adj_ref[...], ab_ref[...],
                  preferred_element_type=jnp.float32)
    o_ref[...] = jnp.maximum(acc, 0.0)


def _pass2_kernel(adj_ref, h_ref, w2_ref, eps_ref, mean_ref,
                  col_ref, row_ref, dmask_ref, o_ref):
    m = jnp.dot(adj_ref[...], h_ref[...],
                preferred_element_type=jnp.float32)
    g = jnp.dot(m, w2_ref[...], preferred_element_type=jnp.float32)
    lvec = jnp.maximum(g, 0.0)
    d = dmask_ref[...]
    p = jnp.where(d > 0.0, jnp.maximum(lvec, THRESH), lvec)
    e = jnp.dot(eps_ref[...], col_ref[...],
                preferred_element_type=jnp.float32)
    tr = jnp.dot(p * e, row_ref[...],
                 preferred_element_type=jnp.float32)
    o_ref[...] = tr + mean_ref[...]


def kernel(input, adj, W_mean, W1, W2):
    n = adj.shape[0]
    wcat = jnp.concatenate([W_mean, W1], axis=1)
    c = wcat.shape[1]
    w2p = jnp.pad(W2, ((0, 0), (0, TRI_PAD - TRI)))
    eps = jax.random.normal(jax.random.key(1234), (n, OUT_F),
                            dtype=jnp.float32)
    col_oh = jnp.asarray(_COL_OH)
    row_oh = jnp.asarray(_ROW_OH)
    dmask = jnp.asarray(_DIAG)

    ab = pl.pallas_call(
        _feat_kernel,
        out_shape=jax.ShapeDtypeStruct((n, c), jnp.float32),
    )(input, wcat)

    r = 400 if n % 400 == 0 else n
    grid = (n // r,)

    mh = pl.pallas_call(
        _pass1_kernel,
        grid=grid,
        in_specs=[pl.BlockSpec((r, n), lambda i: (i, 0)),
                  pl.BlockSpec((n, c), lambda i: (0, 0))],
        out_specs=pl.BlockSpec((r, c), lambda i: (i, 0)),
        out_shape=jax.ShapeDtypeStruct((n, c), jnp.float32),
    )(adj, ab)

    mean = mh[:, :OUT_F]
    h = mh[:, OUT_F:]

    out = pl.pallas_call(
        _pass2_kernel,
        grid=grid,
        in_specs=[pl.BlockSpec((r, n), lambda i: (i, 0)),
                  pl.BlockSpec((n, HID), lambda i: (0, 0)),
                  pl.BlockSpec((HID, TRI_PAD), lambda i: (0, 0)),
                  pl.BlockSpec((r, OUT_F), lambda i: (i, 0)),
                  pl.BlockSpec((r, OUT_F), lambda i: (i, 0)),
                  pl.BlockSpec((OUT_F, TRI_PAD), lambda i: (0, 0)),
                  pl.BlockSpec((TRI_PAD, OUT_F), lambda i: (0, 0)),
                  pl.BlockSpec((1, TRI_PAD), lambda i: (0, 0))],
        out_specs=pl.BlockSpec((r, OUT_F), lambda i: (i, 0)),
        out_shape=jax.ShapeDtypeStruct((n, OUT_F), jnp.float32),
    )(adj, h, w2p, eps, mean, col_oh, row_oh, dmask)
    return out
